# Initial kernel scaffold; baseline (speedup 1.0000x reference)
import functools

import jax
import jax.numpy as jnp
from jax import lax
from jax.experimental import pallas as pl
from jax.experimental.pallas import tpu as pltpu
from jax.experimental.pallas import tpu_sc as plsc

NN = 10000
EE = 320000
DD = 128

NC = 2
NS = 16
NW = NC * NS

EPW = EE // NW
CHUNK = 100
NCHUNK = EPW // CHUNK

N_PAD = 10240
ROWS_PT = N_PAD // NS


def _sc_agg_body(x_hbm, src_hbm, dst_hbm, zero_hbm, out_hbm,
                 agg_sh, s_v, d_v, rows, sem):
    c = lax.axis_index("c")
    s = lax.axis_index("s")
    wid = s * NC + c

    pltpu.sync_copy(zero_hbm.at[pl.ds(s * ROWS_PT, ROWS_PT)],
                    agg_sh.at[pl.ds(s * ROWS_PT, ROWS_PT)])

    pltpu.sync_copy(src_hbm.at[wid], s_v)
    pltpu.sync_copy(dst_hbm.at[wid], d_v)
    plsc.subcore_barrier()

    def chunk_body(i, carry):
        pltpu.async_copy(x_hbm.at[s_v.at[i]], rows, sem).wait()
        pltpu.sync_copy(rows, agg_sh.at[d_v.at[i]], add=True)
        return carry

    lax.fori_loop(0, NCHUNK, chunk_body, 0)

    plsc.subcore_barrier()
    pltpu.sync_copy(agg_sh.at[pl.ds(s * ROWS_PT, ROWS_PT)],
                    out_hbm.at[c, pl.ds(s * ROWS_PT, ROWS_PT)])


_sc_agg = pl.kernel(
    _sc_agg_body,
    out_type=jax.ShapeDtypeStruct((NC, N_PAD, DD), jnp.float32),
    mesh=plsc.VectorSubcoreMesh(core_axis_name="c", subcore_axis_name="s"),
    scratch_types=[
        pltpu.VMEM_SHARED((N_PAD, DD), jnp.float32),
        pltpu.VMEM((NCHUNK, CHUNK), jnp.int32),
        pltpu.VMEM((NCHUNK, CHUNK), jnp.int32),
        pltpu.VMEM((CHUNK, DD), jnp.float32),
        pltpu.SemaphoreType.DMA,
    ],
)


def _mlp_block(x_ref, a0_ref, a1_ref, w1_ref, b1_ref, w2_ref, b2_ref, o_ref,
               *, relu_out):
    h = x_ref[...] + a0_ref[...] + a1_ref[...]
    t = jnp.dot(h, w1_ref[...], preferred_element_type=jnp.float32)
    t = jnp.maximum(t + b1_ref[...], 0.0)
    o = jnp.dot(t, w2_ref[...], preferred_element_type=jnp.float32)
    o = o + b2_ref[...]
    if relu_out:
        o = jnp.maximum(o, 0.0)
    o_ref[...] = o


def _mlp(x, a0, a1, w1, b1, w2, b2, relu_out):
    blk = 1000
    grid = NN // blk
    row_spec = pl.BlockSpec((blk, DD), lambda i: (i, 0))
    w_spec = pl.BlockSpec((DD, DD), lambda i: (0, 0))
    b_spec = pl.BlockSpec((1, DD), lambda i: (0, 0))
    return pl.pallas_call(
        functools.partial(_mlp_block, relu_out=relu_out),
        grid=(grid,),
        in_specs=[row_spec, row_spec, row_spec, w_spec, b_spec, w_spec, b_spec],
        out_specs=row_spec,
        out_shape=jax.ShapeDtypeStruct((NN, DD), jnp.float32),
    )(x, a0, a1, w1, b1.reshape(1, DD), w2, b2.reshape(1, DD))


def kernel(x, edge_index, W1_0, b1_0, W2_0, b2_0, W1_1, b1_1, W2_1, b2_1,
           W1_2, b1_2, W2_2, b2_2):
    src = edge_index[0].reshape(NW, NCHUNK, CHUNK)
    dst = edge_index[1].reshape(NW, NCHUNK, CHUNK)
    zero = jnp.zeros((N_PAD, DD), dtype=jnp.float32)

    h = x
    for (w1, b1, w2, b2, relu_out) in (
        (W1_0, b1_0, W2_0, b2_0, True),
        (W1_1, b1_1, W2_1, b2_1, True),
        (W1_2, b1_2, W2_2, b2_2, False),
    ):
        parts = _sc_agg(h, src, dst, zero)
        h = _mlp(h, parts[0, :NN], parts[1, :NN], w1, b1, w2, b2, relu_out)
    return h


# speedup vs baseline: 6.8954x; 6.8954x over previous
import functools

import jax
import jax.numpy as jnp
from jax import lax
from jax.experimental import pallas as pl
from jax.experimental.pallas import tpu as pltpu
from jax.experimental.pallas import tpu_sc as plsc

NN = 10000
EE = 320000
DD = 128

NC = 2
NS = 16
NW = NC * NS

EPW = EE // NW
CHUNK = 100
NCHUNK = EPW // CHUNK

N_PAD = 10240
ROWS_PT = N_PAD // NS


def _sc_agg_body(x_hbm, src_hbm, dst_hbm, zero_hbm, out_hbm,
                 agg_sh, s_v, d_v, rows, sem):
    c = lax.axis_index("c")
    s = lax.axis_index("s")
    wid = s * NC + c

    pltpu.sync_copy(zero_hbm.at[pl.ds(s * ROWS_PT, ROWS_PT)],
                    agg_sh.at[pl.ds(s * ROWS_PT, ROWS_PT)])

    pltpu.sync_copy(src_hbm.at[wid], s_v)
    pltpu.sync_copy(dst_hbm.at[wid], d_v)
    plsc.subcore_barrier()

    def chunk_body(i, carry):
        pltpu.async_copy(x_hbm.at[s_v.at[i]], rows, sem).wait()
        pltpu.sync_copy(rows, agg_sh.at[d_v.at[i]], add=True)
        return carry

    lax.fori_loop(0, NCHUNK, chunk_body, 0)

    plsc.subcore_barrier()
    pltpu.sync_copy(agg_sh.at[pl.ds(s * ROWS_PT, ROWS_PT)],
                    out_hbm.at[c, pl.ds(s * ROWS_PT, ROWS_PT)])


_sc_agg = pl.kernel(
    _sc_agg_body,
    out_type=jax.ShapeDtypeStruct((NC, N_PAD, DD), jnp.float32),
    mesh=plsc.VectorSubcoreMesh(core_axis_name="c", subcore_axis_name="s"),
    scratch_types=[
        pltpu.VMEM_SHARED((N_PAD, DD), jnp.float32),
        pltpu.VMEM((NCHUNK, CHUNK), jnp.int32),
        pltpu.VMEM((NCHUNK, CHUNK), jnp.int32),
        pltpu.VMEM((CHUNK, DD), jnp.float32),
        pltpu.SemaphoreType.DMA,
    ],
)


def _mlp_block(x_ref, a_ref, w1_ref, b1_ref, w2_ref, b2_ref, o_ref,
               *, relu_out):
    h = x_ref[...] + a_ref[...]
    t = jnp.dot(h, w1_ref[...], preferred_element_type=jnp.float32)
    t = jnp.maximum(t + b1_ref[...], 0.0)
    o = jnp.dot(t, w2_ref[...], preferred_element_type=jnp.float32)
    o = o + b2_ref[...]
    if relu_out:
        o = jnp.maximum(o, 0.0)
    o_ref[...] = o


def _mlp(x, a, w1, b1, w2, b2, relu_out):
    blk = 1000
    grid = NN // blk
    row_spec = pl.BlockSpec((blk, DD), lambda i: (i, 0))
    w_spec = pl.BlockSpec((DD, DD), lambda i: (0, 0))
    b_spec = pl.BlockSpec((1, DD), lambda i: (0, 0))
    return pl.pallas_call(
        functools.partial(_mlp_block, relu_out=relu_out),
        grid=(grid,),
        in_specs=[row_spec, row_spec, w_spec, b_spec, w_spec, b_spec],
        out_specs=row_spec,
        out_shape=jax.ShapeDtypeStruct((NN, DD), jnp.float32),
    )(x, a, w1, b1.reshape(1, DD), w2, b2.reshape(1, DD))


def kernel(x, edge_index, W1_0, b1_0, W2_0, b2_0, W1_1, b1_1, W2_1, b2_1,
           W1_2, b1_2, W2_2, b2_2):
    src = edge_index[0].reshape(NW, NCHUNK, CHUNK)
    dst = edge_index[1].reshape(NW, NCHUNK, CHUNK)
    zero = jnp.zeros((N_PAD, DD), dtype=jnp.float32)

    h = x
    for (w1, b1, w2, b2, relu_out) in (
        (W1_0, b1_0, W2_0, b2_0, True),
        (W1_1, b1_1, W2_1, b2_1, True),
        (W1_2, b1_2, W2_2, b2_2, False),
    ):
        hin = lax.optimization_barrier(h * jnp.float32(1.0))
        parts = _sc_agg(hin, src, dst, zero)
        a = lax.optimization_barrier(parts[0, :NN] + parts[1, :NN])
        h = _mlp(h, a, w1, b1, w2, b2, relu_out)
    return h
